# Initial kernel scaffold; baseline (speedup 1.0000x reference)
import jax
import jax.numpy as jnp
from jax.experimental import pallas as pl


def kernel(x, edge_index, W1, b1, W2, b2, W_ih_f, W_hh_f, b_ih_f, b_hh_f, W_ih_r, W_hh_r, b_ih_r, b_hh_r, W_att, b_att, W_lin, b_lin):
    raise NotImplementedError("write your pallas kernel here")


# speedup vs baseline: 29.1630x; 29.1630x over previous
import functools

import jax
import jax.numpy as jnp
from jax import lax
from jax.experimental import pallas as pl
from jax.experimental.pallas import tpu as pltpu
from jax.experimental.pallas import tpu_sc as plsc

N = 100000
E = 1600000
D_IN = 128
HID = 16
OUT = 64
LSTM_H = 32

NW = 32
LANES = 128
KB = 8
RPW = 392
R_TOT = NW * RPW
E_PAD = R_TOT * LANES
NB = RPW // KB
N_PAD = 100352
STRIPE = N_PAD // 16
ZR = 392

BLK = 4096
GRID = -(-N // BLK)

def _prop_body(h_hbm, src_hbm, dst_hbm, out_hbm, src_v, dst_v, rows_v, zv, acc, sem):
    c = lax.axis_index("c")
    s = lax.axis_index("s")
    w = s * 2 + c
    base = s * STRIPE

    def zrow(i, carry):
        zv[i] = jnp.zeros((HID,), jnp.float32)
        return carry

    lax.fori_loop(0, ZR, zrow, 0)
    for r in range(STRIPE // ZR):
        pltpu.sync_copy(zv, acc.at[pl.ds(base + r * ZR, ZR)])
    plsc.subcore_barrier()

    row0 = w * RPW

    def block(bi, carry):
        rbase = row0 + bi * KB
        pltpu.sync_copy(src_hbm.at[pl.ds(rbase, KB)], src_v)
        pltpu.sync_copy(dst_hbm.at[pl.ds(rbase, KB)], dst_v)
        copies = [
            pltpu.async_copy(h_hbm.at[src_v.at[j]], rows_v.at[j], sem)
            for j in range(KB)
        ]
        for cp in copies:
            cp.wait()
        for j in range(KB):
            pltpu.sync_copy(rows_v.at[j], acc.at[dst_v.at[j]], add=True)
        return carry

    lax.fori_loop(0, NB, block, 0)
    plsc.subcore_barrier()
    pltpu.sync_copy(acc.at[pl.ds(base, STRIPE)], out_hbm.at[c, pl.ds(base, STRIPE)])


def _deg_body(dst_hbm, out_hbm, dst_v, ones_v, zv, acc):
    c = lax.axis_index("c")
    s = lax.axis_index("s")
    w = s * 2 + c
    base = s * STRIPE

    def zchunk(i, carry):
        zv[pl.ds(i * 16, 16)] = jnp.zeros((16,), jnp.float32)
        return carry

    lax.fori_loop(0, ZR // 16, zchunk, 0)
    for j in range(LANES // 16):
        ones_v[pl.ds(j * 16, 16)] = jnp.ones((16,), jnp.float32)
    for r in range(STRIPE // ZR):
        pltpu.sync_copy(zv, acc.at[pl.ds(base + r * ZR, ZR)])
    plsc.subcore_barrier()

    row0 = w * RPW

    def block(bi, carry):
        rbase = row0 + bi * KB
        pltpu.sync_copy(dst_hbm.at[pl.ds(rbase, KB)], dst_v)
        for j in range(KB):
            pltpu.sync_copy(ones_v, acc.at[dst_v.at[j]], add=True)
        return carry

    lax.fori_loop(0, NB, block, 0)
    plsc.subcore_barrier()
    pltpu.sync_copy(acc.at[pl.ds(base, STRIPE)], out_hbm.at[c, pl.ds(base, STRIPE)])


@functools.lru_cache(maxsize=None)
def _prop_kernel():
    mesh = plsc.VectorSubcoreMesh(core_axis_name="c", subcore_axis_name="s")
    return pl.kernel(
        _prop_body,
        mesh=mesh,
        compiler_params=pltpu.CompilerParams(use_tc_tiling_on_sc=False),
        out_type=jax.ShapeDtypeStruct((2, N_PAD, HID), jnp.float32),
        scratch_types=[
            pltpu.VMEM((KB, LANES), jnp.int32),
            pltpu.VMEM((KB, LANES), jnp.int32),
            pltpu.VMEM((KB, LANES, HID), jnp.float32),
            pltpu.VMEM((ZR, HID), jnp.float32),
            pltpu.VMEM_SHARED((N_PAD, HID), jnp.float32),
            pltpu.SemaphoreType.DMA,
        ],
    )


@functools.lru_cache(maxsize=None)
def _deg_kernel():
    mesh = plsc.VectorSubcoreMesh(core_axis_name="c", subcore_axis_name="s")
    return pl.kernel(
        _deg_body,
        mesh=mesh,
        compiler_params=pltpu.CompilerParams(use_tc_tiling_on_sc=False),
        out_type=jax.ShapeDtypeStruct((2, N_PAD), jnp.float32),
        scratch_types=[
            pltpu.VMEM((KB, LANES), jnp.int32),
            pltpu.VMEM((LANES,), jnp.float32),
            pltpu.VMEM((ZR,), jnp.float32),
            pltpu.VMEM_SHARED((N_PAD,), jnp.float32),
        ],
    )


def _prop_call(h, srcr, dstr):
    return _prop_kernel()(h, srcr, dstr)


def _deg_call(dstr):
    return _deg_kernel()(dstr)


def _rowmm(a, w):
    return lax.dot_general(a, w, (((1,), (1,)), ((), ())),
                           preferred_element_type=jnp.float32)


def _k2_body(x_ref, w1_ref, degp_ref, dinv_ref, a0_ref):
    h0 = _rowmm(x_ref[...], w1_ref[...])
    deg = degp_ref[0] + degp_ref[1] + 1.0
    dinv = lax.rsqrt(deg)[:, None]
    dinv_ref[...] = dinv
    a0_ref[...] = h0 * dinv


def _k2(x, W1, degp):
    return pl.pallas_call(
        _k2_body,
        grid=(GRID,),
        in_specs=[
            pl.BlockSpec((BLK, D_IN), lambda i: (i, 0)),
            pl.BlockSpec((HID, D_IN), lambda i: (0, 0)),
            pl.BlockSpec((2, BLK), lambda i: (0, i)),
        ],
        out_specs=[
            pl.BlockSpec((BLK, 1), lambda i: (i, 0)),
            pl.BlockSpec((BLK, HID), lambda i: (i, 0)),
        ],
        out_shape=[
            jax.ShapeDtypeStruct((N, 1), jnp.float32),
            jax.ShapeDtypeStruct((N, HID), jnp.float32),
        ],
    )(x, W1, degp)


def _k3_body(p_ref, a0_ref, dinv_ref, b1_ref, w2_ref, x1_ref, a1_ref):
    dinv = dinv_ref[...]
    ssum = p_ref[0] + p_ref[1] + a0_ref[...]
    x1 = jnp.maximum(ssum * dinv + b1_ref[...], 0.0)
    x1_ref[...] = x1
    a1_ref[...] = _rowmm(x1, w2_ref[...]) * dinv


def _k3(P0, a0, dinv, b1, W2):
    return pl.pallas_call(
        _k3_body,
        grid=(GRID,),
        in_specs=[
            pl.BlockSpec((2, BLK, HID), lambda i: (0, i, 0)),
            pl.BlockSpec((BLK, HID), lambda i: (i, 0)),
            pl.BlockSpec((BLK, 1), lambda i: (i, 0)),
            pl.BlockSpec((HID,), lambda i: (0,)),
            pl.BlockSpec((HID, HID), lambda i: (0, 0)),
        ],
        out_specs=[
            pl.BlockSpec((BLK, HID), lambda i: (i, 0)),
            pl.BlockSpec((BLK, HID), lambda i: (i, 0)),
        ],
        out_shape=[
            jax.ShapeDtypeStruct((N, HID), jnp.float32),
            jax.ShapeDtypeStruct((N, HID), jnp.float32),
        ],
    )(P0, a0, dinv, b1, W2)


def _lstm_cell(xt, h, c, wih, whh, bsum):
    g = _rowmm(xt, wih) + _rowmm(h, whh) + bsum
    i = jax.nn.sigmoid(g[:, 0 * LSTM_H:1 * LSTM_H])
    f = jax.nn.sigmoid(g[:, 1 * LSTM_H:2 * LSTM_H])
    gg = jnp.tanh(g[:, 2 * LSTM_H:3 * LSTM_H])
    o = jax.nn.sigmoid(g[:, 3 * LSTM_H:4 * LSTM_H])
    c = f * c + i * gg
    h = o * jnp.tanh(c)
    return h, c


def _k4_body(p_ref, a1_ref, dinv_ref, b2_ref, x1_ref,
             wihf_ref, whhf_ref, bf_ref, wihr_ref, whhr_ref, br_ref,
             watt_ref, batt_ref, aj_ref):
    dinv = dinv_ref[...]
    ssum = p_ref[0] + p_ref[1] + a1_ref[...]
    x2 = jnp.maximum(ssum * dinv + b2_ref[...], 0.0)
    x1 = x1_ref[...]
    xs = (x1, x2)

    bf = bf_ref[...]
    br = br_ref[...]
    z = jnp.zeros((x1.shape[0], LSTM_H), jnp.float32)
    h, c = _lstm_cell(xs[0], z, z, wihf_ref[...], whhf_ref[...], bf)
    f0 = h
    h, c = _lstm_cell(xs[1], h, c, wihf_ref[...], whhf_ref[...], bf)
    f1 = h
    h, c = _lstm_cell(xs[1], z, z, wihr_ref[...], whhr_ref[...], br)
    r1 = h
    h, c = _lstm_cell(xs[0], h, c, wihr_ref[...], whhr_ref[...], br)
    r0 = h

    watt = watt_ref[...]
    batt = batt_ref[...]
    al0 = _rowmm(jnp.concatenate([f0, r0], axis=1), watt)[:, 0] + batt[0]
    al1 = _rowmm(jnp.concatenate([f1, r1], axis=1), watt)[:, 0] + batt[0]
    m = jnp.maximum(al0, al1)
    e0 = jnp.exp(al0 - m)
    e1 = jnp.exp(al1 - m)
    inv = 1.0 / (e0 + e1)
    xj = (x1 * (e0 * inv)[:, None] + x2 * (e1 * inv)[:, None])
    aj_ref[...] = xj * dinv


def _k4(P1, a1, dinv, b2, x1, W_ih_f, W_hh_f, bsum_f, W_ih_r, W_hh_r, bsum_r,
        W_att, b_att):
    return pl.pallas_call(
        _k4_body,
        grid=(GRID,),
        in_specs=[
            pl.BlockSpec((2, BLK, HID), lambda i: (0, i, 0)),
            pl.BlockSpec((BLK, HID), lambda i: (i, 0)),
            pl.BlockSpec((BLK, 1), lambda i: (i, 0)),
            pl.BlockSpec((HID,), lambda i: (0,)),
            pl.BlockSpec((BLK, HID), lambda i: (i, 0)),
            pl.BlockSpec((4 * LSTM_H, HID), lambda i: (0, 0)),
            pl.BlockSpec((4 * LSTM_H, LSTM_H), lambda i: (0, 0)),
            pl.BlockSpec((4 * LSTM_H,), lambda i: (0,)),
            pl.BlockSpec((4 * LSTM_H, HID), lambda i: (0, 0)),
            pl.BlockSpec((4 * LSTM_H, LSTM_H), lambda i: (0, 0)),
            pl.BlockSpec((4 * LSTM_H,), lambda i: (0,)),
            pl.BlockSpec((1, 2 * LSTM_H), lambda i: (0, 0)),
            pl.BlockSpec((1,), lambda i: (0,)),
        ],
        out_specs=pl.BlockSpec((BLK, HID), lambda i: (i, 0)),
        out_shape=jax.ShapeDtypeStruct((N, HID), jnp.float32),
    )(P1, a1, dinv, b2, x1, W_ih_f, W_hh_f, bsum_f, W_ih_r, W_hh_r, bsum_r,
      W_att, b_att)


def _k5_body(p_ref, aj_ref, dinv_ref, wlin_ref, blin_ref, out_ref):
    dinv = dinv_ref[...]
    xp = (p_ref[0] + p_ref[1] + aj_ref[...]) * dinv
    o = _rowmm(xp, wlin_ref[...]) + blin_ref[...]
    m = jnp.max(o, axis=1, keepdims=True)
    zz = o - m
    lse = jnp.log(jnp.sum(jnp.exp(zz), axis=1, keepdims=True))
    out_ref[...] = zz - lse


def _k5(Pj, aj, dinv, W_lin, b_lin):
    return pl.pallas_call(
        _k5_body,
        grid=(GRID,),
        in_specs=[
            pl.BlockSpec((2, BLK, HID), lambda i: (0, i, 0)),
            pl.BlockSpec((BLK, HID), lambda i: (i, 0)),
            pl.BlockSpec((BLK, 1), lambda i: (i, 0)),
            pl.BlockSpec((OUT, HID), lambda i: (0, 0)),
            pl.BlockSpec((OUT,), lambda i: (0,)),
        ],
        out_specs=pl.BlockSpec((BLK, OUT), lambda i: (i, 0)),
        out_shape=jax.ShapeDtypeStruct((N, OUT), jnp.float32),
    )(Pj, aj, dinv, W_lin, b_lin)


def kernel(x, edge_index, W1, b1, W2, b2, W_ih_f, W_hh_f, b_ih_f, b_hh_f,
           W_ih_r, W_hh_r, b_ih_r, b_hh_r, W_att, b_att, W_lin, b_lin):
    src = edge_index[0]
    dst = edge_index[1]
    pad_src = jnp.zeros((E_PAD - E,), jnp.int32)
    pad_dst = jnp.full((E_PAD - E,), N_PAD - 1, jnp.int32)
    srcr = jnp.concatenate([src, pad_src]).reshape(R_TOT, LANES)
    dstr = jnp.concatenate([dst, pad_dst]).reshape(R_TOT, LANES)

    degp = _deg_call(dstr)
    dinv, a0 = _k2(x, W1, degp)
    P0 = _prop_call(a0, srcr, dstr)
    x1, a1 = _k3(P0, a0, dinv, b1, W2)
    P1 = _prop_call(a1, srcr, dstr)
    aj = _k4(P1, a1, dinv, b2, x1, W_ih_f, W_hh_f, b_ih_f + b_hh_f,
             W_ih_r, W_hh_r, b_ih_r + b_hh_r, W_att, b_att)
    Pj = _prop_call(aj, srcr, dstr)
    return _k5(Pj, aj, dinv, W_lin, b_lin)
